# Initial kernel scaffold; baseline (speedup 1.0000x reference)
import functools

import jax
import jax.numpy as jnp
from jax import lax
from jax.experimental import pallas as pl
from jax.experimental.pallas import tpu as pltpu
from jax.experimental.pallas import tpu_sc as plsc

N = 10000
D = 128
E = 320000

NC = 2
NS = 16
B = 128
NB = 157
PAD_E = NS * NB * B
ACC_ROWS = N + 16
ZROWS = 64
ROWS_PER_TILE_Z = ACC_ROWS // NS
ROWS_PER_TILE_OUT = N // NS


def _make_kernel():
    mesh = plsc.VectorSubcoreMesh(core_axis_name="c", subcore_axis_name="s")

    @functools.partial(
        pl.kernel,
        out_type=jax.ShapeDtypeStruct((NC, N, D), jnp.float32),
        mesh=mesh,
        scratch_types=[
            pltpu.VMEM((NB, B), jnp.int32),
            pltpu.VMEM((NB, B), jnp.int32),
            pltpu.VMEM((B, D), jnp.float32),
            pltpu.VMEM((ZROWS, D), jnp.float32),
            pltpu.VMEM_SHARED((ACC_ROWS, D), jnp.float32),
            pltpu.SemaphoreType.DMA,
        ],
    )
    def spmm2(x_hbm, idx_hbm, out_hbm, dst_idx, src_idx, rows, zbuf, acc, gsem):
        c = lax.axis_index("c")
        s = lax.axis_index("s")

        pltpu.sync_copy(idx_hbm.at[c, 0, s], dst_idx)
        pltpu.sync_copy(idx_hbm.at[c, 1, s], src_idx)

        def zstore(t, carry):
            zbuf[t >> 3, pl.ds((t & 7) * 16, 16)] = jnp.zeros((16,), jnp.float32)
            return carry
        lax.fori_loop(0, ZROWS * 8, zstore, 0)

        zbase = s * ROWS_PER_TILE_Z
        def zcopy(k, carry):
            pltpu.sync_copy(zbuf, acc.at[pl.ds(zbase + k * ZROWS, ZROWS)])
            return carry
        lax.fori_loop(0, ROWS_PER_TILE_Z // ZROWS, zcopy, 0)
        rem = ROWS_PER_TILE_Z % ZROWS
        if rem:
            pltpu.sync_copy(
                zbuf.at[pl.ds(0, rem)],
                acc.at[pl.ds(zbase + (ROWS_PER_TILE_Z // ZROWS) * ZROWS, rem)],
            )

        plsc.subcore_barrier()

        def body(j, carry):
            pltpu.async_copy(x_hbm.at[src_idx.at[j]], rows, gsem).wait()
            pltpu.sync_copy(rows, acc.at[dst_idx.at[j]], add=True)
            return carry
        lax.fori_loop(0, NB, body, 0)

        plsc.subcore_barrier()

        obase = s * ROWS_PER_TILE_OUT
        pltpu.sync_copy(
            acc.at[pl.ds(obase, ROWS_PER_TILE_OUT)],
            out_hbm.at[c, pl.ds(obase, ROWS_PER_TILE_OUT)],
        )

    return spmm2


_spmm2 = _make_kernel()


@jax.jit
def _run(x, adj_t, adj_t2):
    def prep(adj):
        dst = adj[0].astype(jnp.int32)
        src = adj[1].astype(jnp.int32)
        dst = jnp.concatenate([dst, jnp.full((PAD_E - E,), N, jnp.int32)])
        src = jnp.concatenate([src, jnp.zeros((PAD_E - E,), jnp.int32)])
        return jnp.stack([dst, src])

    idx = jnp.stack([prep(adj_t), prep(adj_t2)]).reshape(NC, 2, NS, NB, B)
    out = _spmm2(x, idx)
    return jnp.concatenate([out[0], out[1]], axis=1)


def kernel(x, adj_t, adj_t2):
    return _run(x, adj_t, adj_t2)


# speedup vs baseline: 5.0314x; 5.0314x over previous
import functools

import jax
import jax.numpy as jnp
from jax import lax
from jax.experimental import pallas as pl
from jax.experimental.pallas import tpu as pltpu
from jax.experimental.pallas import tpu_sc as plsc

N = 10000
D = 128
E = 320000

NC = 2
NS = 16
NH = 2
DH = D // NH
B = 128
NB = 157
PAD_E = NS * NB * B
ACC_ROWS = N + 16
ZROWS = 64
RPT = 624


def _make_kernel():
    mesh = plsc.VectorSubcoreMesh(core_axis_name="c", subcore_axis_name="s")

    @functools.partial(
        pl.kernel,
        out_type=jax.ShapeDtypeStruct((NC, NH, N, DH), jnp.float32),
        mesh=mesh,
        scratch_types=[
            pltpu.VMEM((NB, B), jnp.int32),
            pltpu.VMEM((NB, B), jnp.int32),
            pltpu.VMEM((B, DH), jnp.float32),
            pltpu.VMEM((ZROWS, DH), jnp.float32),
            pltpu.VMEM_SHARED((ACC_ROWS, DH), jnp.float32),
            pltpu.SemaphoreType.DMA,
        ],
        compiler_params=pltpu.CompilerParams(use_tc_tiling_on_sc=False),
    )
    def spmm2(x_hbm, idx_hbm, out_hbm, dst_idx, src_idx, rows, zbuf, acc, gsem):
        c = lax.axis_index("c")
        s = lax.axis_index("s")

        pltpu.sync_copy(idx_hbm.at[c, 0, s], dst_idx)
        pltpu.sync_copy(idx_hbm.at[c, 1, s], src_idx)

        def zstore(t, carry):
            zbuf[t >> 2, pl.ds((t & 3) * 16, 16)] = jnp.zeros((16,), jnp.float32)
            return carry
        lax.fori_loop(0, ZROWS * (DH // 16), zstore, 0)

        zbase = s * RPT
        obase = s * RPT
        ztail = ACC_ROWS - NS * RPT
        otail = N - NS * RPT

        for p in range(NH):
            def zcopy(k, carry):
                pltpu.sync_copy(zbuf, acc.at[pl.ds(zbase + k * ZROWS, ZROWS)])
                return carry
            lax.fori_loop(0, RPT // ZROWS, zcopy, 0)
            rem = RPT % ZROWS
            if rem:
                pltpu.sync_copy(
                    zbuf.at[pl.ds(0, rem)],
                    acc.at[pl.ds(zbase + (RPT // ZROWS) * ZROWS, rem)],
                )

            @pl.when(s == NS - 1)
            def _ztail():
                pltpu.sync_copy(
                    zbuf.at[pl.ds(0, ztail)],
                    acc.at[pl.ds(NS * RPT, ztail)],
                )

            plsc.subcore_barrier()

            def body(j, carry):
                pltpu.async_copy(
                    x_hbm.at[p].at[src_idx.at[j]], rows, gsem
                ).wait()
                pltpu.sync_copy(rows, acc.at[dst_idx.at[j]], add=True)
                return carry
            lax.fori_loop(0, NB, body, 0)

            plsc.subcore_barrier()

            pltpu.sync_copy(
                acc.at[pl.ds(obase, RPT)],
                out_hbm.at[c, p].at[pl.ds(obase, RPT)],
            )

            @pl.when(s == NS - 1)
            def _otail():
                pltpu.sync_copy(
                    acc.at[pl.ds(NS * RPT, otail)],
                    out_hbm.at[c, p].at[pl.ds(NS * RPT, otail)],
                )

            plsc.subcore_barrier()

    return spmm2


_spmm2 = _make_kernel()


@jax.jit
def _run(x, adj_t, adj_t2):
    def prep(adj):
        dst = adj[0].astype(jnp.int32)
        src = adj[1].astype(jnp.int32)
        dst = jnp.concatenate([dst, jnp.full((PAD_E - E,), N, jnp.int32)])
        src = jnp.concatenate([src, jnp.zeros((PAD_E - E,), jnp.int32)])
        return jnp.stack([dst, src])

    idx = jnp.stack([prep(adj_t), prep(adj_t2)]).reshape(NC, 2, NS, NB, B)
    xh = jnp.transpose(x.reshape(N, NH, DH), (1, 0, 2))
    out = _spmm2(xh, idx)
    return jnp.concatenate([out[0, 0], out[0, 1], out[1, 0], out[1, 1]], axis=1)


def kernel(x, adj_t, adj_t2):
    return _run(x, adj_t, adj_t2)
